# Initial kernel scaffold; baseline (speedup 1.0000x reference)
import jax
import jax.numpy as jnp
from jax.experimental import pallas as pl


def kernel(x, edge_index, edge_weight, initial_weight, gru_w_ih, gru_w_hh, gru_b_ih, gru_b_hh, lin_w, lin_b):
    raise NotImplementedError("write your pallas kernel here")


# speedup vs baseline: 9.8623x; 9.8623x over previous
import functools

import jax
import jax.numpy as jnp
from jax import lax
from jax.experimental import pallas as pl
from jax.experimental.pallas import tpu as pltpu
from jax.experimental.pallas import tpu_sc as plsc

N_NODES = 10000
D = 128
N_PAD = 10240
E_PAD = 327680
CHUNK = 256
GJ = CHUNK // 128
ROWS_PT = N_PAD // 16
EPT_C = E_PAD // 32
NCH_C = EPT_C // CHUNK
EPT_D = E_PAD // 16
NCH_D = EPT_D // CHUNK


def _evolve_body(iw_ref, wih_ref, whh_ref, bih_ref, bhh_ref, w_ref):
    iw = iw_ref[...]
    xw = jnp.dot(iw, wih_ref[...], preferred_element_type=jnp.float32) + bih_ref[...]
    hw = jnp.dot(iw, whh_ref[...], preferred_element_type=jnp.float32) + bhh_ref[...]
    r = jax.nn.sigmoid(xw[:, :D] + hw[:, :D])
    z = jax.nn.sigmoid(xw[:, D:2 * D] + hw[:, D:2 * D])
    n = jnp.tanh(xw[:, 2 * D:] + r * hw[:, 2 * D:])
    w_ref[...] = (1.0 - z) * n + z * iw


def _xw_body(x_ref, w_ref, o_ref):
    o_ref[...] = jnp.dot(x_ref[...], w_ref[...], preferred_element_type=jnp.float32)


def _post_body(p0_ref, p1_ref, xw_ref, dis_ref, lwt_ref, lb_ref, o_ref):
    dis = dis_ref[...]
    inv = dis * dis
    h = p0_ref[...] + p1_ref[...] + inv * xw_ref[...]
    h = jnp.maximum(h, 0.0)
    o_ref[...] = jnp.dot(h, lwt_ref[...], preferred_element_type=jnp.float32) + lb_ref[...]


def _sc_body(row_hbm, col_hbm, ew_hbm, xw_hbm, part_hbm, dis_hbm,
             acc_sh, deg_sh, dis_v, rowi, coli, ewv, normv, rows, sem):
    c = lax.axis_index("c")
    s = lax.axis_index("s")
    zero16 = jnp.zeros((16,), jnp.float32)

    def zrow(i, carry):
        for j in range(8):
            rows[i, pl.ds(j * 16, 16)] = zero16
        return carry
    lax.fori_loop(0, CHUNK, zrow, 0)

    def zdis(i, carry):
        dis_v[pl.ds(i * 16, 16)] = zero16
        return carry
    lax.fori_loop(0, N_PAD // 16, zdis, 0)

    pltpu.sync_copy(rows.at[pl.ds(0, 256)], acc_sh.at[pl.ds(s * ROWS_PT, 256)])
    pltpu.sync_copy(rows.at[pl.ds(0, 256)],
                    acc_sh.at[pl.ds(s * ROWS_PT + 256, 256)])
    pltpu.sync_copy(rows.at[pl.ds(0, 128)],
                    acc_sh.at[pl.ds(s * ROWS_PT + 512, 128)])
    pltpu.sync_copy(dis_v.at[pl.ds(0, ROWS_PT)],
                    deg_sh.at[pl.ds(s * ROWS_PT, ROWS_PT)])
    plsc.subcore_barrier()

    def deg_chunk(k, carry):
        base = (s * NCH_D + k) * GJ
        pltpu.sync_copy(col_hbm.at[pl.ds(base, GJ)], coli)
        pltpu.sync_copy(ew_hbm.at[pl.ds(base, GJ)], ewv)
        for j in range(GJ):
            pltpu.sync_copy(ewv.at[j], deg_sh.at[coli.at[j]], add=True)
        return carry
    lax.fori_loop(0, NCH_D, deg_chunk, 0)
    plsc.subcore_barrier()

    def disb(i, carry):
        pltpu.sync_copy(deg_sh.at[pl.ds(i * 128, 128)],
                        normv.at[pl.ds(0, 128)])
        for l in range(8):
            d = normv[pl.ds(l * 16, 16)] + 1.0
            xi = lax.bitcast_convert_type(d, jnp.int32)
            yi = jnp.int32(0x5F3759DF) - lax.shift_right_arithmetic(xi, 1)
            y = lax.bitcast_convert_type(yi, jnp.float32)
            for _ in range(3):
                y = y * (1.5 - 0.5 * d * y * y)
            dis_v[pl.ds(i * 128 + l * 16, 16)] = y
        return carry
    lax.fori_loop(0, N_PAD // 128, disb, 0)

    @pl.when(c == 0)
    def _():
        pltpu.sync_copy(dis_v.at[pl.ds(s * ROWS_PT, ROWS_PT)],
                        dis_hbm.at[pl.ds(s * ROWS_PT, ROWS_PT)])

    def main_chunk(k, carry):
        base = c * (E_PAD // 2 // 128) + (s * NCH_C + k) * GJ
        pltpu.sync_copy(row_hbm.at[pl.ds(base, GJ)], rowi)
        pltpu.sync_copy(col_hbm.at[pl.ds(base, GJ)], coli)
        pltpu.sync_copy(ew_hbm.at[pl.ds(base, GJ)], ewv)
        for j in range(GJ):
            pltpu.make_async_copy(xw_hbm.at[rowi.at[j]],
                                  rows.at[pl.ds(j * 128, 128)], sem).start()
        for j in range(GJ):
            for l in range(8):
                ir = rowi[j, pl.ds(l * 16, 16)]
                ic = coli[j, pl.ds(l * 16, 16)]
                w = ewv[j, pl.ds(l * 16, 16)]
                nr = plsc.load_gather(dis_v, [ir])
                nc = plsc.load_gather(dis_v, [ic])
                normv[pl.ds(j * 128 + l * 16, 16)] = nr * w * nc
        for j in range(GJ):
            pltpu.make_async_copy(xw_hbm.at[rowi.at[j]],
                                  rows.at[pl.ds(j * 128, 128)], sem).wait()

        def scale(e16, carry2):
            nv16 = normv[pl.ds(e16 * 16, 16)]
            for u in range(16):
                e = e16 * 16 + u
                nv = nv16[u]
                for j in range(8):
                    rows[e, pl.ds(j * 16, 16)] = rows[e, pl.ds(j * 16, 16)] * nv
            return carry2
        lax.fori_loop(0, CHUNK // 16, scale, 0)

        for j in range(GJ):
            pltpu.sync_copy(rows.at[pl.ds(j * 128, 128)],
                            acc_sh.at[coli.at[j]], add=True)
        return carry
    lax.fori_loop(0, NCH_C, main_chunk, 0)
    plsc.subcore_barrier()

    pltpu.sync_copy(acc_sh.at[pl.ds(s * ROWS_PT, ROWS_PT)],
                    part_hbm.at[c].at[pl.ds(s * ROWS_PT, ROWS_PT)])


_sc_call = functools.partial(
    pl.kernel,
    out_type=[
        jax.ShapeDtypeStruct((2, N_PAD, D), jnp.float32),
        jax.ShapeDtypeStruct((N_PAD,), jnp.float32),
    ],
    mesh=plsc.VectorSubcoreMesh(core_axis_name="c", subcore_axis_name="s"),
    compiler_params=pltpu.CompilerParams(needs_layout_passes=False),
    scratch_types=[
        pltpu.VMEM_SHARED((N_PAD, D), jnp.float32),
        pltpu.VMEM_SHARED((N_PAD,), jnp.float32),
        pltpu.VMEM((N_PAD,), jnp.float32),
        pltpu.VMEM((GJ, 128), jnp.int32),
        pltpu.VMEM((GJ, 128), jnp.int32),
        pltpu.VMEM((GJ, 128), jnp.float32),
        pltpu.VMEM((CHUNK,), jnp.float32),
        pltpu.VMEM((CHUNK, D), jnp.float32),
        pltpu.SemaphoreType.DMA,
    ],
)(_sc_body)


@jax.jit
def _run(x, edge_index, edge_weight, initial_weight,
         gru_w_ih, gru_w_hh, gru_b_ih, gru_b_hh, lin_w, lin_b):
    row = edge_index[0].astype(jnp.int32)
    col = edge_index[1].astype(jnp.int32)
    ew = edge_weight.astype(jnp.float32)
    e = row.shape[0]
    row2d = jnp.zeros((E_PAD,), jnp.int32).at[:e].set(row).reshape(E_PAD // 128, 128)
    col2d = jnp.zeros((E_PAD,), jnp.int32).at[:e].set(col).reshape(E_PAD // 128, 128)
    ew2d = jnp.zeros((E_PAD,), jnp.float32).at[:e].set(ew).reshape(E_PAD // 128, 128)
    n = x.shape[0]
    x_p = jnp.zeros((N_PAD, D), jnp.float32).at[:n].set(x)

    w_evo = pl.pallas_call(
        _evolve_body,
        out_shape=jax.ShapeDtypeStruct((D, D), jnp.float32),
    )(initial_weight, gru_w_ih.T, gru_w_hh.T,
      gru_b_ih.reshape(1, 3 * D), gru_b_hh.reshape(1, 3 * D))

    blk = 1024
    nblk = N_PAD // blk
    xw = pl.pallas_call(
        _xw_body,
        grid=(nblk,),
        in_specs=[
            pl.BlockSpec((blk, D), lambda i: (i, 0)),
            pl.BlockSpec((D, D), lambda i: (0, 0)),
        ],
        out_specs=pl.BlockSpec((blk, D), lambda i: (i, 0)),
        out_shape=jax.ShapeDtypeStruct((N_PAD, D), jnp.float32),
    )(x_p, w_evo)

    part, deg = _sc_call(row2d, col2d, ew2d, xw)

    n_t = lin_w.shape[0]
    out = pl.pallas_call(
        _post_body,
        grid=(nblk,),
        in_specs=[
            pl.BlockSpec((blk, D), lambda i: (i, 0)),
            pl.BlockSpec((blk, D), lambda i: (i, 0)),
            pl.BlockSpec((blk, D), lambda i: (i, 0)),
            pl.BlockSpec((blk, 1), lambda i: (i, 0)),
            pl.BlockSpec((D, n_t), lambda i: (0, 0)),
            pl.BlockSpec((1, n_t), lambda i: (0, 0)),
        ],
        out_specs=pl.BlockSpec((blk, n_t), lambda i: (i, 0)),
        out_shape=jax.ShapeDtypeStruct((N_PAD, n_t), jnp.float32),
    )(part[0], part[1], xw, deg.reshape(N_PAD, 1), lin_w.T, lin_b.reshape(1, n_t))
    return out[:n]


def kernel(x, edge_index, edge_weight, initial_weight,
           gru_w_ih, gru_w_hh, gru_b_ih, gru_b_hh, lin_w, lin_b):
    return _run(x, edge_index, edge_weight, initial_weight,
                gru_w_ih, gru_w_hh, gru_b_ih, gru_b_hh, lin_w, lin_b)
